# baseline (speedup 1.0000x reference)
import jax
import jax.numpy as jnp
from jax.experimental import pallas as pl
from jax.experimental.pallas import tpu as pltpu

_DT = 512
_H = _DT // 2


def _spec_add_kernel(labels_ref, xa_ref, xb_ref, emb_ref, o_ref):
    e = emb_ref[0, 0, :]
    o_ref[:, :_H, :] = xa_ref[...] + e[None, :_H, None]
    o_ref[:, _H:, :] = xb_ref[...] + e[None, _H:, None]


def kernel(x, spec_labels, table):
    B, D, S = x.shape
    grid = (B, D // _DT)
    table3 = table.reshape(table.shape[0], 1, D)
    grid_spec = pltpu.PrefetchScalarGridSpec(
        num_scalar_prefetch=1,
        grid=grid,
        in_specs=[
            pl.BlockSpec((1, _H, S), lambda b, d, labels: (b, 2 * d, 0)),
            pl.BlockSpec((1, _H, S), lambda b, d, labels: (b, 2 * d + 1, 0)),
            pl.BlockSpec((1, 1, _DT), lambda b, d, labels: (labels[b], 0, d)),
        ],
        out_specs=pl.BlockSpec((1, _DT, S), lambda b, d, labels: (b, d, 0)),
    )
    return pl.pallas_call(
        _spec_add_kernel,
        grid_spec=grid_spec,
        out_shape=jax.ShapeDtypeStruct((B, D, S), x.dtype),
        compiler_params=pltpu.CompilerParams(
            dimension_semantics=("parallel", "parallel"),
            vmem_limit_bytes=64 * 1024 * 1024,
        ),
    )(spec_labels.astype(jnp.int32), x, x, table3)
